# initial kernel scaffold (unmeasured)
import jax
import jax.numpy as jnp
from jax import lax
from jax.experimental import pallas as pl
from jax.experimental.pallas import tpu as pltpu

N_DEV = 32
LOG2_N = 5
B, SQ, SKV, H_LOC, DH = 2, 128, 128, 4, 64
D_MODEL = 512
HD = H_LOC * DH
BLK = 64


def kernel(x, Wq, K_ext, V_ext, Wo):
    my_pos = lax.axis_index("i")
    idx = jnp.full((1,), my_pos, dtype=jnp.int32)

    def body(idx_ref, x_ref, wq_ref, k_ref, v_ref, wo_ref, out_ref,
             acc_ref, recv_ref, send_sems, recv_sems):
        pos = idx_ref[0]

        wq = wq_ref[...].astype(jnp.bfloat16)
        wo = wo_ref[...].astype(jnp.bfloat16)

        rows = lax.broadcasted_iota(jnp.int32, (SQ, SKV), 0) // BLK
        cols = lax.broadcasted_iota(jnp.int32, (SQ, SKV), 1) // BLK
        mask = cols <= rows

        for b in range(B):
            x_b = x_ref[b].astype(jnp.bfloat16)
            q_b = lax.dot_general(
                x_b, wq, (((1,), (0,)), ((), ())),
                preferred_element_type=jnp.float32,
            )
            q_b = (q_b * 0.125).astype(jnp.bfloat16)
            ctx_parts = []
            for h in range(H_LOC):
                qh = q_b[:, h * DH:(h + 1) * DH]
                kh = k_ref[b, :, h, :].astype(jnp.bfloat16)
                vh = v_ref[b, :, h, :].astype(jnp.bfloat16)
                s = lax.dot_general(
                    qh, kh, (((1,), (1,)), ((), ())),
                    preferred_element_type=jnp.float32,
                )
                s = jnp.where(mask, s, -1e9)
                m = jnp.max(s, axis=1, keepdims=True)
                w = jnp.exp(s - m)
                w = w / jnp.sum(w, axis=1, keepdims=True)
                ctx_parts.append(lax.dot_general(
                    w.astype(jnp.bfloat16), vh, (((1,), (0,)), ((), ())),
                    preferred_element_type=jnp.float32,
                ))
            ctx_b = jnp.concatenate(ctx_parts, axis=1).astype(jnp.bfloat16)
            acc_ref[b, :, :] = lax.dot_general(
                ctx_b, wo, (((1,), (0,)), ((), ())),
                preferred_element_type=jnp.float32,
            )

        for r in range(LOG2_N):
            partner = pos ^ (1 << r)
            rdma = pltpu.make_async_remote_copy(
                src_ref=acc_ref,
                dst_ref=recv_ref.at[r],
                send_sem=send_sems.at[r],
                recv_sem=recv_sems.at[r],
                device_id=(partner,),
                device_id_type=pl.DeviceIdType.MESH,
            )
            rdma.start()
            rdma.wait()
            acc_ref[...] = acc_ref[...] + recv_ref[r]

        out_ref[...] = acc_ref[...]

    grid_spec = pltpu.PrefetchScalarGridSpec(
        num_scalar_prefetch=1,
        grid=(1,),
        in_specs=[
            pl.BlockSpec((B, SQ, D_MODEL), lambda i, s: (0, 0, 0)),
            pl.BlockSpec((D_MODEL, HD), lambda i, s: (0, s[0])),
            pl.BlockSpec((B, SKV, H_LOC, DH), lambda i, s: (0, 0, 0, 0)),
            pl.BlockSpec((B, SKV, H_LOC, DH), lambda i, s: (0, 0, 0, 0)),
            pl.BlockSpec((HD, D_MODEL), lambda i, s: (s[0], 0)),
        ],
        out_specs=pl.BlockSpec((B, SQ, D_MODEL), lambda i, s: (0, 0, 0)),
        scratch_shapes=[
            pltpu.VMEM((B, SQ, D_MODEL), jnp.float32),
            pltpu.VMEM((LOG2_N, B, SQ, D_MODEL), jnp.float32),
            pltpu.SemaphoreType.DMA((LOG2_N,)),
            pltpu.SemaphoreType.DMA((LOG2_N,)),
        ],
    )
    return pl.pallas_call(
        body,
        grid_spec=grid_spec,
        out_shape=jax.ShapeDtypeStruct((B, SQ, D_MODEL), jnp.float32),
        compiler_params=pltpu.CompilerParams(
            dimension_semantics=("arbitrary",),
        ),
    )(idx, x, Wq, K_ext, V_ext, Wo)


# baseline (device time: 76617 ns/iter reference)
import jax
import jax.numpy as jnp
from jax import lax
from jax.experimental import pallas as pl
from jax.experimental.pallas import tpu as pltpu

N_DEV = 32
LOG2_N = 5
B, SQ, SKV, H_LOC, DH = 2, 128, 128, 4, 64
D_MODEL = 512
HD = H_LOC * DH
BLK = 64


def kernel(x, Wq, K_ext, V_ext, Wo):
    my_pos = lax.axis_index("i")
    idx = jnp.full((1,), my_pos, dtype=jnp.int32)

    def body(idx_ref, x_ref, wq_ref, k_ref, v_ref, wo_ref, out_ref,
             acc_ref, recv_ref, send_sems, recv_sems):
        pos = idx_ref[0]

        wq = wq_ref[...].astype(jnp.bfloat16)
        wo = wo_ref[...].astype(jnp.bfloat16)

        rows = lax.broadcasted_iota(jnp.int32, (SQ, SKV), 0) // BLK
        cols = lax.broadcasted_iota(jnp.int32, (SQ, SKV), 1) // BLK
        mask = cols <= rows

        for b in range(B):
            x_b = x_ref[b].astype(jnp.bfloat16)
            q_b = lax.dot_general(
                x_b, wq, (((1,), (0,)), ((), ())),
                preferred_element_type=jnp.float32,
            )
            q_b = (q_b * 0.125).astype(jnp.bfloat16)
            ctx_parts = []
            for h in range(H_LOC):
                qh = q_b[:, h * DH:(h + 1) * DH]
                kh = k_ref[b, :, h, :].astype(jnp.bfloat16)
                vh = v_ref[b, :, h, :].astype(jnp.bfloat16)
                s = lax.dot_general(
                    qh, kh, (((1,), (1,)), ((), ())),
                    preferred_element_type=jnp.float32,
                )
                s = jnp.where(mask, s, -1e9)
                m = jnp.max(s, axis=1, keepdims=True)
                w = jnp.exp(s - m)
                w = w / jnp.sum(w, axis=1, keepdims=True)
                ctx_parts.append(lax.dot_general(
                    w.astype(jnp.bfloat16), vh, (((1,), (0,)), ((), ())),
                    preferred_element_type=jnp.float32,
                ))
            ctx_b = jnp.concatenate(ctx_parts, axis=1).astype(jnp.bfloat16)
            acc_ref[b, :, :] = lax.dot_general(
                ctx_b, wo, (((1,), (0,)), ((), ())),
                preferred_element_type=jnp.float32,
            )

        for r in range(LOG2_N):
            partner = pos ^ (1 << r)
            rdma = pltpu.make_async_remote_copy(
                src_ref=acc_ref,
                dst_ref=recv_ref.at[r],
                send_sem=send_sems.at[r],
                recv_sem=recv_sems.at[r],
                device_id=partner,
                device_id_type=pl.DeviceIdType.LOGICAL,
            )
            rdma.start()
            rdma.wait()
            acc_ref[...] = acc_ref[...] + recv_ref[r]

        out_ref[...] = acc_ref[...]

    grid_spec = pltpu.PrefetchScalarGridSpec(
        num_scalar_prefetch=1,
        grid=(1,),
        in_specs=[
            pl.BlockSpec((B, SQ, D_MODEL), lambda i, s: (0, 0, 0)),
            pl.BlockSpec((D_MODEL, HD), lambda i, s: (0, s[0])),
            pl.BlockSpec((B, SKV, H_LOC, DH), lambda i, s: (0, 0, 0, 0)),
            pl.BlockSpec((B, SKV, H_LOC, DH), lambda i, s: (0, 0, 0, 0)),
            pl.BlockSpec((HD, D_MODEL), lambda i, s: (s[0], 0)),
        ],
        out_specs=pl.BlockSpec((B, SQ, D_MODEL), lambda i, s: (0, 0, 0)),
        scratch_shapes=[
            pltpu.VMEM((B, SQ, D_MODEL), jnp.float32),
            pltpu.VMEM((LOG2_N, B, SQ, D_MODEL), jnp.float32),
            pltpu.SemaphoreType.DMA((LOG2_N,)),
            pltpu.SemaphoreType.DMA((LOG2_N,)),
        ],
    )
    return pl.pallas_call(
        body,
        grid_spec=grid_spec,
        out_shape=jax.ShapeDtypeStruct((B, SQ, D_MODEL), jnp.float32),
        compiler_params=pltpu.CompilerParams(
            dimension_semantics=("arbitrary",),
        ),
    )(idx, x, Wq, K_ext, V_ext, Wo)


# device time: 57263 ns/iter; 1.3380x vs baseline; 1.3380x over previous
import jax
import jax.numpy as jnp
from jax import lax
from jax.experimental import pallas as pl
from jax.experimental.pallas import tpu as pltpu

N_DEV = 32
LOG2_N = 5
B, SQ, SKV, H_LOC, DH = 2, 128, 128, 4, 64
D_MODEL = 512
HD = H_LOC * DH
BLK = 64


def kernel(x, Wq, K_ext, V_ext, Wo):
    my_pos = lax.axis_index("i")
    idx = jnp.full((1,), my_pos, dtype=jnp.int32)

    def body(idx_ref, x_ref, wq_ref, k_ref, v_ref, wo_ref, out_ref,
             acc_ref, send_ref, recv_ref, send_sems, recv_sems):
        pos = idx_ref[0]

        wq = wq_ref[...].astype(jnp.bfloat16)
        wo = wo_ref[...].astype(jnp.bfloat16)

        rows = lax.broadcasted_iota(jnp.int32, (SQ, SKV), 0) // BLK
        cols = lax.broadcasted_iota(jnp.int32, (SQ, SKV), 1) // BLK
        mask = cols <= rows

        for b in range(B):
            x_b = x_ref[b].astype(jnp.bfloat16)
            q_b = lax.dot_general(
                x_b, wq, (((1,), (0,)), ((), ())),
                preferred_element_type=jnp.float32,
            )
            q_b = (q_b * 0.125).astype(jnp.bfloat16)
            ctx_parts = []
            for h in range(H_LOC):
                qh = q_b[:, h * DH:(h + 1) * DH]
                kh = k_ref[b, :, h, :].astype(jnp.bfloat16)
                vh = v_ref[b, :, h, :].astype(jnp.bfloat16)
                s = lax.dot_general(
                    qh, kh, (((1,), (1,)), ((), ())),
                    preferred_element_type=jnp.float32,
                )
                s = jnp.where(mask, s, -1e9)
                m = jnp.max(s, axis=1, keepdims=True)
                w = jnp.exp(s - m)
                w = w / jnp.sum(w, axis=1, keepdims=True)
                ctx_parts.append(lax.dot_general(
                    w.astype(jnp.bfloat16), vh, (((1,), (0,)), ((), ())),
                    preferred_element_type=jnp.float32,
                ))
            ctx_b = jnp.concatenate(ctx_parts, axis=1).astype(jnp.bfloat16)
            acc_ref[b, :, :] = lax.dot_general(
                ctx_b, wo, (((1,), (0,)), ((), ())),
                preferred_element_type=jnp.float32,
            )

        for r in range(LOG2_N):
            partner = pos ^ (1 << r)
            send_ref[...] = acc_ref[...].astype(jnp.bfloat16)
            rdma = pltpu.make_async_remote_copy(
                src_ref=send_ref,
                dst_ref=recv_ref.at[r],
                send_sem=send_sems.at[r],
                recv_sem=recv_sems.at[r],
                device_id=partner,
                device_id_type=pl.DeviceIdType.LOGICAL,
            )
            rdma.start()
            rdma.wait()
            acc_ref[...] = acc_ref[...] + recv_ref[r].astype(jnp.float32)

        out_ref[...] = acc_ref[...]

    grid_spec = pltpu.PrefetchScalarGridSpec(
        num_scalar_prefetch=1,
        grid=(1,),
        in_specs=[
            pl.BlockSpec((B, SQ, D_MODEL), lambda i, s: (0, 0, 0)),
            pl.BlockSpec((D_MODEL, HD), lambda i, s: (0, s[0])),
            pl.BlockSpec((B, SKV, H_LOC, DH), lambda i, s: (0, 0, 0, 0)),
            pl.BlockSpec((B, SKV, H_LOC, DH), lambda i, s: (0, 0, 0, 0)),
            pl.BlockSpec((HD, D_MODEL), lambda i, s: (s[0], 0)),
        ],
        out_specs=pl.BlockSpec((B, SQ, D_MODEL), lambda i, s: (0, 0, 0)),
        scratch_shapes=[
            pltpu.VMEM((B, SQ, D_MODEL), jnp.float32),
            pltpu.VMEM((B, SQ, D_MODEL), jnp.bfloat16),
            pltpu.VMEM((LOG2_N, B, SQ, D_MODEL), jnp.bfloat16),
            pltpu.SemaphoreType.DMA((LOG2_N,)),
            pltpu.SemaphoreType.DMA((LOG2_N,)),
        ],
    )
    return pl.pallas_call(
        body,
        grid_spec=grid_spec,
        out_shape=jax.ShapeDtypeStruct((B, SQ, D_MODEL), jnp.float32),
        compiler_params=pltpu.CompilerParams(
            dimension_semantics=("arbitrary",),
        ),
    )(idx, x, Wq, K_ext, V_ext, Wo)


# device time: 48398 ns/iter; 1.5831x vs baseline; 1.1832x over previous
import jax
import jax.numpy as jnp
from jax import lax
from jax.experimental import pallas as pl
from jax.experimental.pallas import tpu as pltpu

N_DEV = 32
LOG2_N = 5
B, SQ, SKV, H_LOC, DH = 2, 128, 128, 4, 64
D_MODEL = 512
HD = H_LOC * DH
BLK = 64


def kernel(x, Wq, K_ext, V_ext, Wo):
    my_pos = lax.axis_index("i")
    idx = jnp.full((1,), my_pos, dtype=jnp.int32)

    def body(idx_ref, x_ref, wq_ref, k_ref, v_ref, wo_ref, out_ref,
             acc_ref, send_ref, recv_ref, send_sems, recv_sems):
        pos = idx_ref[0]

        barrier_sem = pltpu.get_barrier_semaphore()
        for r in range(LOG2_N):
            pl.semaphore_signal(
                barrier_sem, inc=1,
                device_id=pos ^ (1 << r),
                device_id_type=pl.DeviceIdType.LOGICAL,
            )

        wq = wq_ref[...].astype(jnp.bfloat16)
        wo = wo_ref[...].astype(jnp.bfloat16)

        rows = lax.broadcasted_iota(jnp.int32, (SQ, SKV), 0) // BLK
        cols = lax.broadcasted_iota(jnp.int32, (SQ, SKV), 1) // BLK
        mask = cols <= rows

        for b in range(B):
            x_b = x_ref[b].astype(jnp.bfloat16)
            q_b = lax.dot_general(
                x_b, wq, (((1,), (0,)), ((), ())),
                preferred_element_type=jnp.float32,
            )
            q_b = (q_b * 0.125).astype(jnp.bfloat16)
            ctx_parts = []
            for h in range(H_LOC):
                qh = q_b[:, h * DH:(h + 1) * DH]
                kh = k_ref[b, :, h, :].astype(jnp.bfloat16)
                vh = v_ref[b, :, h, :].astype(jnp.bfloat16)
                s = lax.dot_general(
                    qh, kh, (((1,), (1,)), ((), ())),
                    preferred_element_type=jnp.float32,
                )
                s = jnp.where(mask, s, -1e9)
                m = jnp.max(s, axis=1, keepdims=True)
                w = jnp.exp(s - m)
                w = w / jnp.sum(w, axis=1, keepdims=True)
                ctx_parts.append(lax.dot_general(
                    w.astype(jnp.bfloat16), vh, (((1,), (0,)), ((), ())),
                    preferred_element_type=jnp.float32,
                ))
            ctx_b = jnp.concatenate(ctx_parts, axis=1).astype(jnp.bfloat16)
            acc_ref[b, :, :] = lax.dot_general(
                ctx_b, wo, (((1,), (0,)), ((), ())),
                preferred_element_type=jnp.float32,
            )

        pl.semaphore_wait(barrier_sem, LOG2_N)

        for r in range(LOG2_N):
            partner = pos ^ (1 << r)
            send_ref[...] = acc_ref[...].astype(jnp.bfloat16)
            rdma = pltpu.make_async_remote_copy(
                src_ref=send_ref,
                dst_ref=recv_ref.at[r],
                send_sem=send_sems.at[r],
                recv_sem=recv_sems.at[r],
                device_id=partner,
                device_id_type=pl.DeviceIdType.LOGICAL,
            )
            rdma.start()
            rdma.wait()
            acc_ref[...] = acc_ref[...] + recv_ref[r].astype(jnp.float32)

        out_ref[...] = acc_ref[...]

    grid_spec = pltpu.PrefetchScalarGridSpec(
        num_scalar_prefetch=1,
        grid=(1,),
        in_specs=[
            pl.BlockSpec((B, SQ, D_MODEL), lambda i, s: (0, 0, 0)),
            pl.BlockSpec((D_MODEL, HD), lambda i, s: (0, s[0])),
            pl.BlockSpec((B, SKV, H_LOC, DH), lambda i, s: (0, 0, 0, 0)),
            pl.BlockSpec((B, SKV, H_LOC, DH), lambda i, s: (0, 0, 0, 0)),
            pl.BlockSpec((HD, D_MODEL), lambda i, s: (s[0], 0)),
        ],
        out_specs=pl.BlockSpec((B, SQ, D_MODEL), lambda i, s: (0, 0, 0)),
        scratch_shapes=[
            pltpu.VMEM((B, SQ, D_MODEL), jnp.float32),
            pltpu.VMEM((B, SQ, D_MODEL), jnp.bfloat16),
            pltpu.VMEM((LOG2_N, B, SQ, D_MODEL), jnp.bfloat16),
            pltpu.SemaphoreType.DMA((LOG2_N,)),
            pltpu.SemaphoreType.DMA((LOG2_N,)),
        ],
    )
    return pl.pallas_call(
        body,
        grid_spec=grid_spec,
        out_shape=jax.ShapeDtypeStruct((B, SQ, D_MODEL), jnp.float32),
        compiler_params=pltpu.CompilerParams(
            dimension_semantics=("arbitrary",),
            collective_id=0,
        ),
    )(idx, x, Wq, K_ext, V_ext, Wo)


# device time: 41893 ns/iter; 1.8289x vs baseline; 1.1553x over previous
import jax
import jax.numpy as jnp
from jax import lax
from jax.experimental import pallas as pl
from jax.experimental.pallas import tpu as pltpu

N_DEV = 32
LOG2_N = 5
B, SQ, SKV, H_LOC, DH = 2, 128, 128, 4, 64
D_MODEL = 512
HD = H_LOC * DH
BLK = 64


def kernel(x, Wq, K_ext, V_ext, Wo):
    my_pos = lax.axis_index("i")
    idx = jnp.full((1,), my_pos, dtype=jnp.int32)

    def body(idx_ref, x_ref, wq_ref, k_ref, v_ref, wo_ref, out_ref,
             acc_ref, send_ref, recv_ref, send_sems, recv_sems):
        pos = idx_ref[0]

        barrier_sem = pltpu.get_barrier_semaphore()
        for r in range(LOG2_N):
            pl.semaphore_signal(
                barrier_sem, inc=1,
                device_id=pos ^ (1 << r),
                device_id_type=pl.DeviceIdType.LOGICAL,
            )

        wq = wq_ref[...].astype(jnp.bfloat16)
        wo = wo_ref[...].astype(jnp.bfloat16)

        rows = lax.broadcasted_iota(jnp.int32, (SQ, SKV), 0) // BLK
        cols = lax.broadcasted_iota(jnp.int32, (SQ, SKV), 1) // BLK
        mask = cols <= rows

        for b in range(B):
            x_b = x_ref[b].astype(jnp.bfloat16)
            q_b = lax.dot_general(
                x_b, wq, (((1,), (0,)), ((), ())),
                preferred_element_type=jnp.float32,
            )
            q_b = (q_b * 0.125).astype(jnp.bfloat16)
            ctx_parts = []
            for h in range(H_LOC):
                qh = q_b[:, h * DH:(h + 1) * DH]
                kh = k_ref[b, :, h, :].astype(jnp.bfloat16)
                vh = v_ref[b, :, h, :].astype(jnp.bfloat16)
                s = lax.dot_general(
                    qh, kh, (((1,), (1,)), ((), ())),
                    preferred_element_type=jnp.float32,
                )
                s = jnp.where(mask, s, -1e9)
                m = jnp.max(s, axis=1, keepdims=True)
                w = jnp.exp(s - m)
                w = w / jnp.sum(w, axis=1, keepdims=True)
                ctx_parts.append(lax.dot_general(
                    w.astype(jnp.bfloat16), vh, (((1,), (0,)), ((), ())),
                    preferred_element_type=jnp.float32,
                ))
            ctx_b = jnp.concatenate(ctx_parts, axis=1).astype(jnp.bfloat16)
            acc_ref[b, :, :] = lax.dot_general(
                ctx_b, wo, (((1,), (0,)), ((), ())),
                preferred_element_type=jnp.float32,
            )

        pl.semaphore_wait(barrier_sem, LOG2_N)

        def mk(c, r):
            return pltpu.make_async_remote_copy(
                src_ref=send_ref.at[c, r],
                dst_ref=recv_ref.at[c, r],
                send_sem=send_sems.at[c, r],
                recv_sem=recv_sems.at[c, r],
                device_id=pos ^ (1 << r),
                device_id_type=pl.DeviceIdType.LOGICAL,
            )

        rdmas = {}
        for c in range(B):
            send_ref[c, 0, :, :] = acc_ref[c].astype(jnp.bfloat16)
            rdmas[c, 0] = mk(c, 0)
            rdmas[c, 0].start()
        for r in range(LOG2_N):
            for c in range(B):
                rdmas[c, r].wait_recv()
                new = acc_ref[c] + recv_ref[c, r].astype(jnp.float32)
                acc_ref[c, :, :] = new
                if r + 1 < LOG2_N:
                    send_ref[c, r + 1, :, :] = new.astype(jnp.bfloat16)
                    rdmas[c, r + 1] = mk(c, r + 1)
                    rdmas[c, r + 1].start()
        for c in range(B):
            for r in range(LOG2_N):
                rdmas[c, r].wait_send()

        out_ref[...] = acc_ref[...]

    grid_spec = pltpu.PrefetchScalarGridSpec(
        num_scalar_prefetch=1,
        grid=(1,),
        in_specs=[
            pl.BlockSpec((B, SQ, D_MODEL), lambda i, s: (0, 0, 0)),
            pl.BlockSpec((D_MODEL, HD), lambda i, s: (0, s[0])),
            pl.BlockSpec((B, SKV, H_LOC, DH), lambda i, s: (0, 0, 0, 0)),
            pl.BlockSpec((B, SKV, H_LOC, DH), lambda i, s: (0, 0, 0, 0)),
            pl.BlockSpec((HD, D_MODEL), lambda i, s: (s[0], 0)),
        ],
        out_specs=pl.BlockSpec((B, SQ, D_MODEL), lambda i, s: (0, 0, 0)),
        scratch_shapes=[
            pltpu.VMEM((B, SQ, D_MODEL), jnp.float32),
            pltpu.VMEM((B, LOG2_N, SQ, D_MODEL), jnp.bfloat16),
            pltpu.VMEM((B, LOG2_N, SQ, D_MODEL), jnp.bfloat16),
            pltpu.SemaphoreType.DMA((B, LOG2_N)),
            pltpu.SemaphoreType.DMA((B, LOG2_N)),
        ],
    )
    return pl.pallas_call(
        body,
        grid_spec=grid_spec,
        out_shape=jax.ShapeDtypeStruct((B, SQ, D_MODEL), jnp.float32),
        compiler_params=pltpu.CompilerParams(
            dimension_semantics=("arbitrary",),
            collective_id=0,
        ),
    )(idx, x, Wq, K_ext, V_ext, Wo)


# device time: 33993 ns/iter; 2.2539x vs baseline; 1.2324x over previous
import jax
import jax.numpy as jnp
from jax import lax
from jax.experimental import pallas as pl
from jax.experimental.pallas import tpu as pltpu

N_DEV = 32
B, SQ, SKV, H_LOC, DH = 2, 128, 128, 4, 64
D_MODEL = 512
HD = H_LOC * DH
BLK = 64
SL = (B * SQ) // N_DEV


def kernel(x, Wq, K_ext, V_ext, Wo):
    my_pos = lax.axis_index("i")
    idx = jnp.full((1,), my_pos, dtype=jnp.int32)

    def body(idx_ref, x_ref, wq_ref, k_ref, v_ref, wo_ref, out_ref,
             acc_ref, p1_send, p1_recv, ag_send, ag_recv,
             p1_send_sems, p1_recv_sems, ag_send_sems, ag_recv_sems,
             dummy_sem):
        pos = idx_ref[0]

        barrier_sem = pltpu.get_barrier_semaphore()
        for k in range(1, N_DEV):
            pl.semaphore_signal(
                barrier_sem, inc=1,
                device_id=pos ^ k,
                device_id_type=pl.DeviceIdType.LOGICAL,
            )

        wq = wq_ref[...].astype(jnp.bfloat16)
        wo = wo_ref[...].astype(jnp.bfloat16)

        rows = lax.broadcasted_iota(jnp.int32, (SQ, SKV), 0) // BLK
        cols = lax.broadcasted_iota(jnp.int32, (SQ, SKV), 1) // BLK
        mask = cols <= rows

        for b in range(B):
            x_b = x_ref[b].astype(jnp.bfloat16)
            q_b = lax.dot_general(
                x_b, wq, (((1,), (0,)), ((), ())),
                preferred_element_type=jnp.float32,
            )
            q_b = (q_b * 0.125).astype(jnp.bfloat16)
            ctx_parts = []
            for h in range(H_LOC):
                qh = q_b[:, h * DH:(h + 1) * DH]
                kh = k_ref[b, :, h, :].astype(jnp.bfloat16)
                vh = v_ref[b, :, h, :].astype(jnp.bfloat16)
                s = lax.dot_general(
                    qh, kh, (((1,), (1,)), ((), ())),
                    preferred_element_type=jnp.float32,
                )
                s = jnp.where(mask, s, -1e9)
                m = jnp.max(s, axis=1, keepdims=True)
                w = jnp.exp(s - m)
                w = w / jnp.sum(w, axis=1, keepdims=True)
                ctx_parts.append(lax.dot_general(
                    w.astype(jnp.bfloat16), vh, (((1,), (0,)), ((), ())),
                    preferred_element_type=jnp.float32,
                ))
            ctx_b = jnp.concatenate(ctx_parts, axis=1).astype(jnp.bfloat16)
            partial_b = lax.dot_general(
                ctx_b, wo, (((1,), (0,)), ((), ())),
                preferred_element_type=jnp.float32,
            )
            acc_ref[16 * b:16 * (b + 1), :, :] = partial_b.reshape(16, SL, D_MODEL)

        for k in range(1, N_DEV):
            dest = pos ^ k
            p1_send[k - 1, :, :] = (
                acc_ref[pl.ds(dest, 1), :, :].astype(jnp.bfloat16)
                .reshape(SL, D_MODEL)
            )

        pl.semaphore_wait(barrier_sem, N_DEV - 1)

        p1 = {}
        for k in range(1, N_DEV):
            p1[k] = pltpu.make_async_remote_copy(
                src_ref=p1_send.at[k - 1],
                dst_ref=p1_recv.at[pos],
                send_sem=p1_send_sems.at[k - 1],
                recv_sem=p1_recv_sems.at[k - 1],
                device_id=pos ^ k,
                device_id_type=pl.DeviceIdType.LOGICAL,
            )
            p1[k].start()

        p1_recv[pl.ds(pos, 1), :, :] = acc_ref[pl.ds(pos, 1), :, :].astype(jnp.bfloat16)
        for k in range(1, N_DEV):
            recv_wait = pltpu.make_async_remote_copy(
                src_ref=p1_send.at[0],
                dst_ref=p1_recv.at[pos ^ k],
                send_sem=dummy_sem.at[0],
                recv_sem=p1_recv_sems.at[k - 1],
                device_id=pos,
                device_id_type=pl.DeviceIdType.LOGICAL,
            )
            recv_wait.wait_recv()
        red = jnp.sum(p1_recv[...].astype(jnp.float32), axis=0)

        red_bf = red.astype(jnp.bfloat16)
        ag_send[...] = red_bf
        ag = {}
        for k in range(1, N_DEV):
            ag[k] = pltpu.make_async_remote_copy(
                src_ref=ag_send,
                dst_ref=ag_recv.at[pos],
                send_sem=ag_send_sems.at[k - 1],
                recv_sem=ag_recv_sems.at[k - 1],
                device_id=pos ^ k,
                device_id_type=pl.DeviceIdType.LOGICAL,
            )
            ag[k].start()
        ag_recv[pl.ds(pos, 1), :, :] = red_bf.reshape(1, SL, D_MODEL)
        for k in range(1, N_DEV):
            recv_wait = pltpu.make_async_remote_copy(
                src_ref=ag_send,
                dst_ref=ag_recv.at[pos ^ k],
                send_sem=dummy_sem.at[0],
                recv_sem=ag_recv_sems.at[k - 1],
                device_id=pos,
                device_id_type=pl.DeviceIdType.LOGICAL,
            )
            recv_wait.wait_recv()

        for s in range(N_DEV):
            b, r0 = s // 16, SL * (s % 16)
            out_ref[b, r0:r0 + SL, :] = ag_recv[s].astype(jnp.float32)

        for k in range(1, N_DEV):
            p1[k].wait_send()
            ag[k].wait_send()

    grid_spec = pltpu.PrefetchScalarGridSpec(
        num_scalar_prefetch=1,
        grid=(1,),
        in_specs=[
            pl.BlockSpec((B, SQ, D_MODEL), lambda i, s: (0, 0, 0)),
            pl.BlockSpec((D_MODEL, HD), lambda i, s: (0, s[0])),
            pl.BlockSpec((B, SKV, H_LOC, DH), lambda i, s: (0, 0, 0, 0)),
            pl.BlockSpec((B, SKV, H_LOC, DH), lambda i, s: (0, 0, 0, 0)),
            pl.BlockSpec((HD, D_MODEL), lambda i, s: (s[0], 0)),
        ],
        out_specs=pl.BlockSpec((B, SQ, D_MODEL), lambda i, s: (0, 0, 0)),
        scratch_shapes=[
            pltpu.VMEM((N_DEV, SL, D_MODEL), jnp.float32),
            pltpu.VMEM((N_DEV - 1, SL, D_MODEL), jnp.bfloat16),
            pltpu.VMEM((N_DEV, SL, D_MODEL), jnp.bfloat16),
            pltpu.VMEM((SL, D_MODEL), jnp.bfloat16),
            pltpu.VMEM((N_DEV, SL, D_MODEL), jnp.bfloat16),
            pltpu.SemaphoreType.DMA((N_DEV - 1,)),
            pltpu.SemaphoreType.DMA((N_DEV - 1,)),
            pltpu.SemaphoreType.DMA((N_DEV - 1,)),
            pltpu.SemaphoreType.DMA((N_DEV - 1,)),
            pltpu.SemaphoreType.DMA((1,)),
        ],
    )
    return pl.pallas_call(
        body,
        grid_spec=grid_spec,
        out_shape=jax.ShapeDtypeStruct((B, SQ, D_MODEL), jnp.float32),
        compiler_params=pltpu.CompilerParams(
            dimension_semantics=("arbitrary",),
            collective_id=0,
        ),
    )(idx, x, Wq, K_ext, V_ext, Wo)
